# Initial kernel scaffold; baseline (speedup 1.0000x reference)
import jax
import jax.numpy as jnp
from jax.experimental import pallas as pl


def kernel(x, table):
    raise NotImplementedError("write your pallas kernel here")


# speedup vs baseline: 1.4379x; 1.4379x over previous
import functools

import jax
import jax.numpy as jnp
from jax import lax
from jax.experimental import pallas as pl
from jax.experimental.pallas import tpu as pltpu
from jax.experimental.pallas import tpu_sc as plsc

BATCH = 16384
N_FIELDS = 26
EMB = 32
TOTAL = BATCH * N_FIELDS
NW = 32
CHUNK = 128
NCH = TOTAL // (NW * CHUNK)

_mesh = plsc.VectorSubcoreMesh(core_axis_name="c", subcore_axis_name="s")


@functools.partial(
    pl.kernel,
    mesh=_mesh,
    out_type=jax.ShapeDtypeStruct((NW, NCH, CHUNK, EMB), jnp.float32),
    scratch_types=[
        pltpu.VMEM((NCH, CHUNK), jnp.int32),
        pltpu.VMEM((CHUNK, EMB), jnp.float32),
        pltpu.SemaphoreType.DMA,
    ],
    compiler_params=pltpu.CompilerParams(use_tc_tiling_on_sc=False),
)
def _emb_gather(idx_hbm, table_hbm, out_hbm, idx_v, rows_v, sem):
    wid = lax.axis_index("s") * 2 + lax.axis_index("c")
    pltpu.sync_copy(idx_hbm.at[wid], idx_v)

    def body(j, carry):
        pltpu.async_copy(table_hbm.at[idx_v.at[j]], rows_v, sem).wait()
        pltpu.sync_copy(rows_v, out_hbm.at[wid, j])
        return carry

    lax.fori_loop(0, NCH, body, 0)


def kernel(x, table):
    idx = x.reshape(NW, NCH, CHUNK).astype(jnp.int32)
    out = _emb_gather(idx, table)
    return out.reshape(BATCH, N_FIELDS, EMB)


# speedup vs baseline: 1.5750x; 1.0954x over previous
import functools

import jax
import jax.numpy as jnp
from jax import lax
from jax.experimental import pallas as pl
from jax.experimental.pallas import tpu as pltpu
from jax.experimental.pallas import tpu_sc as plsc

BATCH = 16384
N_FIELDS = 26
EMB = 32
TOTAL = BATCH * N_FIELDS
NW = 32
CHUNK = 128
NCH = TOTAL // (NW * CHUNK)
D = 8
G = 4

_mesh = plsc.VectorSubcoreMesh(core_axis_name="c", subcore_axis_name="s")


@functools.partial(
    pl.kernel,
    mesh=_mesh,
    out_type=jax.ShapeDtypeStruct((NW, NCH, CHUNK, EMB), jnp.float32),
    scratch_types=[
        pltpu.VMEM((NCH, CHUNK), jnp.int32),
        pltpu.VMEM((D, CHUNK, EMB), jnp.float32),
        pltpu.SemaphoreType.DMA((D,)),
        pltpu.SemaphoreType.DMA((D,)),
    ],
    compiler_params=pltpu.CompilerParams(use_tc_tiling_on_sc=False),
)
def _emb_gather(idx_hbm, table_hbm, out_hbm, idx_v, rows_v, gsem, ssem):
    wid = lax.axis_index("s") * 2 + lax.axis_index("c")
    pltpu.sync_copy(idx_hbm.at[wid], idx_v)

    def start_gather(b, j):
        pltpu.async_copy(table_hbm.at[idx_v.at[j]], rows_v.at[b], gsem.at[b])

    def wait_gather(b, j):
        pltpu.make_async_copy(
            table_hbm.at[idx_v.at[j]], rows_v.at[b], gsem.at[b]).wait()

    def start_store(b, j):
        pltpu.async_copy(rows_v.at[b], out_hbm.at[wid, j], ssem.at[b])

    def wait_store(b, j):
        pltpu.make_async_copy(
            rows_v.at[b], out_hbm.at[wid, j], ssem.at[b]).wait()

    for j in range(G):
        start_gather(j % D, j)

    def group(g, carry):
        for b in range(D):
            j = g * D + b
            c_pre = j + G
            bp = (b + G) % D

            @pl.when(c_pre < NCH)
            def _():
                @pl.when(c_pre >= D)
                def _():
                    wait_store(bp, c_pre - D)
                start_gather(bp, c_pre)

            wait_gather(b, j)
            start_store(b, j)
        return carry

    lax.fori_loop(0, NCH // D, group, 0)

    for c in range(NCH - D, NCH):
        wait_store(c % D, c)


def kernel(x, table):
    idx = x.reshape(NW, NCH, CHUNK).astype(jnp.int32)
    out = _emb_gather(idx, table)
    return out.reshape(BATCH, N_FIELDS, EMB)
